# baseline (speedup 1.0000x reference)
import functools

import jax
import jax.numpy as jnp
from jax import lax
from jax.experimental import pallas as pl
from jax.experimental.pallas import tpu as pltpu
from jax.experimental.pallas import tpu_sc as plsc


@functools.cache
def _make_gather(V, D, B):
    info = plsc.get_sparse_core_info()
    NC, NS = info.num_cores, info.num_subcores
    NW = NC * NS
    assert B % (8 * NW) == 0, (B, NW)
    b_per_w = B // NW
    mesh = plsc.VectorSubcoreMesh(core_axis_name="c", subcore_axis_name="s")

    @functools.partial(
        pl.kernel,
        mesh=mesh,
        out_type=jax.ShapeDtypeStruct((B, D), jnp.float32),
        scratch_types=[
            pltpu.VMEM((b_per_w,), jnp.int32),
            pltpu.VMEM((b_per_w, D), jnp.float32),
            pltpu.SemaphoreType.DMA,
        ],
        compiler_params=pltpu.CompilerParams(use_tc_tiling_on_sc=False),
    )
    def k(table_hbm, idx_hbm, out_hbm, idx_v, rows_v, sem):
        wid = lax.axis_index("s") * NC + lax.axis_index("c")
        base = wid * b_per_w
        pltpu.sync_copy(idx_hbm.at[pl.ds(base, b_per_w)], idx_v)
        pltpu.async_copy(table_hbm.at[idx_v], rows_v, sem).wait()
        pltpu.sync_copy(rows_v, out_hbm.at[pl.ds(base, b_per_w)])

    return k


def kernel(domains, table):
    (B,) = domains.shape
    V, D = table.shape
    idx = domains.astype(jnp.int32)
    return _make_gather(V, D, B)(table, idx)


# speedup vs baseline: 1.4469x; 1.4469x over previous
import functools

import jax
import jax.numpy as jnp
from jax import lax
from jax.experimental import pallas as pl
from jax.experimental.pallas import tpu as pltpu
from jax.experimental.pallas import tpu_sc as plsc


@functools.cache
def _make_gather(V, D, B):
    info = plsc.get_sparse_core_info()
    NC, NS = info.num_cores, info.num_subcores
    NW = NC * NS
    L = 16
    assert B % (8 * NW) == 0 and V % 8 == 0, (B, V, NW)
    b_per_w = B // NW
    NG = b_per_w // L
    mesh = plsc.VectorSubcoreMesh(core_axis_name="c", subcore_axis_name="s")

    @functools.partial(
        pl.kernel,
        mesh=mesh,
        out_type=jax.ShapeDtypeStruct((B, D), jnp.float32),
        scratch_types=[
            pltpu.VMEM((b_per_w,), jnp.int32),
            pltpu.VMEM((b_per_w, D), jnp.float32),
            pltpu.SemaphoreType.DMA,
        ],
        compiler_params=pltpu.CompilerParams(use_tc_tiling_on_sc=True),
    )
    def k(table_hbm, idx_hbm, out_hbm, idx_v, rows, sem):
        wid = lax.axis_index("s") * NC + lax.axis_index("c")
        base = wid * b_per_w
        pltpu.sync_copy(idx_hbm.at[pl.ds(base, b_per_w)], idx_v)

        def issue_group(g):
            r = idx_v[pl.ds(g * L, L)]
            t = lax.shift_right_logical(r, 3)
            s = lax.rem(r, 8)
            for j in range(L):
                pltpu.async_copy(
                    table_hbm.at[t[j], s[j]], rows.at[g * L + j], sem)

        def wait_group():
            pltpu.make_async_copy(
                out_hbm.at[pl.ds(0, L)], rows.at[pl.ds(0, L)], sem).wait()

        issue_group(0)

        def body(g, _):
            issue_group(g + 1)
            wait_group()
            return 0

        lax.fori_loop(0, NG - 1, body, 0)
        wait_group()
        pltpu.sync_copy(rows, out_hbm.at[pl.ds(base, b_per_w)])

    return k


def kernel(domains, table):
    (B,) = domains.shape
    V, D = table.shape
    t3 = table.reshape(V // 8, 8, D)
    idx = domains.astype(jnp.int32)
    return _make_gather(V, D, B)(t3, idx)


# speedup vs baseline: 2.3579x; 1.6297x over previous
import functools

import jax
import jax.numpy as jnp
from jax import lax
from jax.experimental import pallas as pl
from jax.experimental.pallas import tpu as pltpu
from jax.experimental.pallas import tpu_sc as plsc


@functools.cache
def _make_gather(V, D, B):
    info = plsc.get_sparse_core_info()
    NC, NS = info.num_cores, info.num_subcores
    L = 16
    NW = NC * NS
    assert D == NW, (D, NW)
    CB = 8192
    assert B % CB == 0 and CB % L == 0
    mesh = plsc.VectorSubcoreMesh(core_axis_name="c", subcore_axis_name="s")

    @functools.partial(
        pl.kernel,
        mesh=mesh,
        out_type=jax.ShapeDtypeStruct((D, B), jnp.float32),
        scratch_types=[
            pltpu.VMEM((V,), jnp.float32),
            pltpu.VMEM((CB,), jnp.int32),
            pltpu.VMEM((CB,), jnp.float32),
        ],
        compiler_params=pltpu.CompilerParams(
            use_tc_tiling_on_sc=True, needs_layout_passes=False),
    )
    def k(tableT_hbm, idx_hbm, outT_hbm, col_v, idx_v, out_v):
        d = lax.axis_index("s") * NC + lax.axis_index("c")
        pltpu.sync_copy(tableT_hbm.at[d], col_v)
        for c in range(B // CB):
            pltpu.sync_copy(idx_hbm.at[pl.ds(c * CB, CB)], idx_v)

            def body(g, _):
                vec = idx_v[pl.ds(g * L, L)]
                out_v[pl.ds(g * L, L)] = plsc.load_gather(col_v, [vec])
                return 0

            lax.fori_loop(0, CB // L, body, 0)
            pltpu.sync_copy(out_v, outT_hbm.at[d, pl.ds(c * CB, CB)])

    return k


def kernel(domains, table):
    (B,) = domains.shape
    V, D = table.shape
    idx = domains.astype(jnp.int32)
    outT = _make_gather(V, D, B)(table.T, idx)
    return outT.T


# speedup vs baseline: 2.8688x; 1.2167x over previous
import functools

import jax
import jax.numpy as jnp
from jax import lax
from jax.experimental import pallas as pl
from jax.experimental.pallas import tpu as pltpu
from jax.experimental.pallas import tpu_sc as plsc


@functools.cache
def _make_gather(V, D, B):
    info = plsc.get_sparse_core_info()
    NC, NS = info.num_cores, info.num_subcores
    L = 16
    NW = NC * NS
    assert D == NW, (D, NW)
    CB = 4096
    NCH = B // CB
    assert B % CB == 0 and CB % L == 0
    mesh = plsc.VectorSubcoreMesh(core_axis_name="c", subcore_axis_name="s")

    @functools.partial(
        pl.kernel,
        mesh=mesh,
        out_type=jax.ShapeDtypeStruct((D, B), jnp.float32),
        scratch_types=[
            pltpu.VMEM((V,), jnp.float32),
            pltpu.VMEM((B,), jnp.int32),
            pltpu.VMEM((2, CB), jnp.float32),
            pltpu.SemaphoreType.DMA,
            pltpu.SemaphoreType.DMA,
            pltpu.SemaphoreType.DMA,
            pltpu.SemaphoreType.DMA,
        ],
        compiler_params=pltpu.CompilerParams(
            use_tc_tiling_on_sc=True, needs_layout_passes=False),
    )
    def k(tableT_hbm, idx_hbm, outT_hbm, col_v, idx_v, out_v, sc, si, so0, so1):
        d = lax.axis_index("s") * NC + lax.axis_index("c")
        col_cp = pltpu.async_copy(tableT_hbm.at[d], col_v, sc)
        idx_cp = pltpu.async_copy(idx_hbm, idx_v, si)
        col_cp.wait()
        idx_cp.wait()
        out_cps = [None, None]
        out_sems = (so0, so1)
        for c in range(NCH):
            bi = c & 1

            @plsc.parallel_loop(c * CB, (c + 1) * CB, step=L, unroll=8)
            def body(i):
                vec = idx_v[pl.ds(i, L)]
                out_v[bi, pl.ds(i - c * CB, L)] = plsc.load_gather(col_v, [vec])

            if out_cps[bi] is not None:
                out_cps[bi].wait()
            out_cps[bi] = pltpu.async_copy(
                out_v.at[bi], outT_hbm.at[d, pl.ds(c * CB, CB)], out_sems[bi])
        out_cps[0].wait()
        out_cps[1].wait()

    return k


def kernel(domains, table):
    (B,) = domains.shape
    V, D = table.shape
    idx = domains.astype(jnp.int32)
    outT = _make_gather(V, D, B)(table.T, idx)
    return outT.T


# speedup vs baseline: 2.8779x; 1.0032x over previous
import functools

import jax
import jax.numpy as jnp
from jax import lax
from jax.experimental import pallas as pl
from jax.experimental.pallas import tpu as pltpu
from jax.experimental.pallas import tpu_sc as plsc


@functools.cache
def _make_gather(V, D, B):
    info = plsc.get_sparse_core_info()
    NC, NS = info.num_cores, info.num_subcores
    L = 16
    NW = NC * NS
    assert D == NW, (D, NW)
    CB = 4096
    NCH = B // CB
    assert B % CB == 0 and CB % L == 0
    mesh = plsc.VectorSubcoreMesh(core_axis_name="c", subcore_axis_name="s")

    @functools.partial(
        pl.kernel,
        mesh=mesh,
        out_type=jax.ShapeDtypeStruct((D, B), jnp.float32),
        scratch_types=[
            pltpu.VMEM((V,), jnp.float32),
            pltpu.VMEM((B,), jnp.int32),
            pltpu.VMEM((2, CB), jnp.float32),
            pltpu.SemaphoreType.DMA,
            pltpu.SemaphoreType.DMA,
            pltpu.SemaphoreType.DMA,
            pltpu.SemaphoreType.DMA,
        ],
        compiler_params=pltpu.CompilerParams(
            use_tc_tiling_on_sc=True, needs_layout_passes=False,
            skip_device_barrier=True),
    )
    def k(tableT_hbm, idx_hbm, outT_hbm, col_v, idx_v, out_v, sc, si, so0, so1):
        d = lax.axis_index("s") * NC + lax.axis_index("c")
        col_cp = pltpu.async_copy(tableT_hbm.at[d], col_v, sc)
        idx_cp = pltpu.async_copy(idx_hbm, idx_v, si)
        col_cp.wait()
        idx_cp.wait()
        out_cps = [None, None]
        out_sems = (so0, so1)
        for c in range(NCH):
            bi = c & 1

            @plsc.parallel_loop(c * CB, (c + 1) * CB, step=L, unroll=8)
            def body(i):
                vec = idx_v[pl.ds(i, L)]
                out_v[bi, pl.ds(i - c * CB, L)] = plsc.load_gather(col_v, [vec])

            if out_cps[bi] is not None:
                out_cps[bi].wait()
            out_cps[bi] = pltpu.async_copy(
                out_v.at[bi], outT_hbm.at[d, pl.ds(c * CB, CB)], out_sems[bi])
        out_cps[0].wait()
        out_cps[1].wait()

    return k


def kernel(domains, table):
    (B,) = domains.shape
    V, D = table.shape
    idx = domains.astype(jnp.int32)
    outT = _make_gather(V, D, B)(table.T, idx)
    return outT.T
